# baseline (speedup 1.0000x reference)
import functools

import jax
import jax.numpy as jnp
from jax import lax
from jax.experimental import pallas as pl
from jax.experimental.pallas import tpu as pltpu
from jax.experimental.pallas import tpu_sc as plsc

EMBED = 32
BATCH = 16384
NUM_PACKED = 1000000 * EMBED // 128

_info = plsc.get_sparse_core_info()
_NC, _NS = _info.num_cores, _info.num_subcores
_NW = _NC * _NS
_BPW = BATCH // _NW
_BURST = 64


_CHUNK = 128
_NCHUNK = _BPW // _CHUNK


def _sc_gather(ut2, it2, urow, irow):
    mesh = plsc.VectorSubcoreMesh(core_axis_name="c", subcore_axis_name="s")

    @functools.partial(
        pl.kernel,
        mesh=mesh,
        out_type=[
            jax.ShapeDtypeStruct((BATCH, 128), jnp.float32),
            jax.ShapeDtypeStruct((BATCH, 128), jnp.float32),
        ],
        scratch_types=[
            pltpu.VMEM((_NCHUNK, _CHUNK), jnp.int32),
            pltpu.VMEM((_NCHUNK, _CHUNK), jnp.int32),
            pltpu.VMEM((3, _CHUNK, 128), jnp.float32),
            pltpu.SemaphoreType.DMA,
            pltpu.SemaphoreType.DMA,
            pltpu.SemaphoreType.DMA,
            pltpu.SemaphoreType.DMA,
        ],
    )
    def k(ut_hbm, it_hbm, ur_hbm, ir_hbm, gu_out, gi_out,
          urow_v, irow_v, ring, sem0, sem1, sem2, wsem):
        sems = [sem0, sem1, sem2]
        wid = lax.axis_index("s") * _NC + lax.axis_index("c")
        base = wid * _BPW
        pltpu.sync_copy(ur_hbm.at[wid], urow_v)
        pltpu.sync_copy(ir_hbm.at[wid], irow_v)

        def fire(s):
            tab, rows = (ut_hbm, urow_v) if s < _NCHUNK else (it_hbm, irow_v)
            return pltpu.async_copy(tab.at[rows.at[s % _NCHUNK]],
                                    ring.at[s % 3], sems[s % 3])

        def writeback(s):
            out = gu_out if s < _NCHUNK else gi_out
            pos = base + (s % _NCHUNK) * _CHUNK
            return pltpu.async_copy(ring.at[s % 3],
                                    out.at[pl.ds(pos, _CHUNK)], wsem)

        cps, wbs = {}, []
        cps[0] = fire(0)
        cps[1] = fire(1)
        for s in range(2 * _NCHUNK):
            cps[s].wait()
            wbs.append(writeback(s))
            if s + 2 < 2 * _NCHUNK:
                if wbs and s >= 1:
                    wbs[s - 1].wait()
                cps[s + 2] = fire(s + 2)
        wbs[2 * _NCHUNK - 3].wait()
        wbs[2 * _NCHUNK - 2].wait()
        wbs[2 * _NCHUNK - 1].wait()

    return k(ut2, it2, urow, irow)


def _mlp_body(gu_ref, gi_ref, ul_ref, il_ref, w1u_ref, w1i_ref, b1_ref,
              w2_ref, b2_ref, w3_ref, b3_ref, out_ref):
    blk = gu_ref.shape[0]
    slot = jax.lax.broadcasted_iota(jnp.int32, (blk, 128), 1) // EMBED
    mu = (slot == ul_ref[...]).astype(jnp.float32)
    mi = (slot == il_ref[...]).astype(jnp.float32)
    u = gu_ref[...] * mu
    v = gi_ref[...] * mi
    h = (jnp.dot(u, w1u_ref[...], preferred_element_type=jnp.float32)
         + jnp.dot(v, w1i_ref[...], preferred_element_type=jnp.float32)
         + b1_ref[...])
    h = jnp.maximum(h, 0.0)
    h2 = jnp.dot(h, w2_ref[...], preferred_element_type=jnp.float32) + b2_ref[...]
    h2 = jnp.maximum(h2, 0.0)
    z = jnp.sum(h2 * w3_ref[...], axis=1) + b3_ref[0, 0]
    out_ref[...] = 1.0 / (1.0 + jnp.exp(-z))


def _tc_mlp(gu, gi, ulane, ilane, w1u4, w1i4, b1r, w2t, b2r, w3r, b3r):
    blk = 2048
    grid = (BATCH // blk,)
    full = lambda shape: pl.BlockSpec(shape, lambda i: (0,) * len(shape))
    return pl.pallas_call(
        _mlp_body,
        grid=grid,
        in_specs=[
            pl.BlockSpec((blk, 128), lambda i: (i, 0)),
            pl.BlockSpec((blk, 128), lambda i: (i, 0)),
            pl.BlockSpec((blk, 1), lambda i: (i, 0)),
            pl.BlockSpec((blk, 1), lambda i: (i, 0)),
            full((128, 64)),
            full((128, 64)),
            full((1, 64)),
            full((64, EMBED)),
            full((1, EMBED)),
            full((1, EMBED)),
            full((1, 1)),
        ],
        out_specs=pl.BlockSpec((blk,), lambda i: (i,)),
        out_shape=jax.ShapeDtypeStruct((BATCH,), jnp.float32),
    )(gu, gi, ulane, ilane, w1u4, w1i4, b1r, w2t, b2r, w3r, b3r)


def kernel(batch_data, user_table, item_table, W1, b1, W2, b2, W3, b3):
    uidx = batch_data[:, 0]
    iidx = batch_data[:, 1]
    ut2 = user_table.reshape(NUM_PACKED, 128)
    it2 = item_table.reshape(NUM_PACKED, 128)
    urow = (uidx // 4).reshape(_NW, _NCHUNK, _CHUNK)
    irow = (iidx // 4).reshape(_NW, _NCHUNK, _CHUNK)
    ulane = (uidx % 4).reshape(BATCH, 1)
    ilane = (iidx % 4).reshape(BATCH, 1)
    gu, gi = _sc_gather(ut2, it2, urow, irow)
    w1t = W1.T
    w1u4 = jnp.tile(w1t[:EMBED], (4, 1))
    w1i4 = jnp.tile(w1t[EMBED:], (4, 1))
    return _tc_mlp(gu, gi, ulane, ilane, w1u4, w1i4, b1.reshape(1, 64),
                   W2.T, b2.reshape(1, EMBED), W3, b3.reshape(1, 1))


# speedup vs baseline: 2.1032x; 2.1032x over previous
import functools

import jax
import jax.numpy as jnp
from jax import lax
from jax.experimental import pallas as pl
from jax.experimental.pallas import tpu as pltpu
from jax.experimental.pallas import tpu_sc as plsc

EMBED = 32
BATCH = 16384
NUM_SLABS = 1000000 // 8

_info = plsc.get_sparse_core_info()
_NC, _NS = _info.num_cores, _info.num_subcores
_NW = _NC * _NS
_BPW = BATCH // _NW
_CH = 32
_NSTEP = _BPW // _CH


def _sc_gather(ut3, it3, uslab, ulane, islab, ilane):
    mesh = plsc.VectorSubcoreMesh(core_axis_name="c", subcore_axis_name="s")

    @functools.partial(
        pl.kernel,
        mesh=mesh,
        compiler_params=pltpu.CompilerParams(needs_layout_passes=False),
        out_type=[
            jax.ShapeDtypeStruct((BATCH // 4, 128), jnp.float32),
            jax.ShapeDtypeStruct((BATCH // 4, 128), jnp.float32),
        ],
        scratch_types=[
            pltpu.VMEM((_BPW,), jnp.int32),
            pltpu.VMEM((_BPW,), jnp.int32),
            pltpu.VMEM((_BPW,), jnp.int32),
            pltpu.VMEM((_BPW,), jnp.int32),
            pltpu.VMEM((2, _CH, 8, EMBED), jnp.float32),
            pltpu.VMEM((2, _CH // 4, 128), jnp.float32),
            pltpu.SemaphoreType.DMA,
            pltpu.SemaphoreType.DMA,
            pltpu.SemaphoreType.DMA,
        ],
    )
    def k(ut_hbm, it_hbm, us_hbm, ul_hbm, is_hbm, il_hbm, gu_out, gi_out,
          ulane_v, ilane_v, us_v, is_v, ring, obuf, sem0, sem1, wsem):
        sems = [sem0, sem1]
        wid = lax.axis_index("s") * _NC + lax.axis_index("c")
        base = wid * _BPW
        pltpu.sync_copy(us_hbm.at[pl.ds(base, _BPW)], us_v)
        pltpu.sync_copy(is_hbm.at[pl.ds(base, _BPW)], is_v)
        pltpu.sync_copy(ul_hbm.at[pl.ds(base, _BPW)], ulane_v)
        pltpu.sync_copy(il_hbm.at[pl.ds(base, _BPW)], ilane_v)
        iota16 = lax.iota(jnp.int32, 16)

        def fire(s):
            tab, slabs = (ut_hbm, us_v) if s < _NSTEP else (it_hbm, is_v)
            c = s % _NSTEP
            slot = s % 2

            def body(g, carry):
                slab16 = slabs[pl.ds(c * _CH + g * 16, 16)]
                for r in range(16):
                    sl = jnp.max(jnp.where(iota16 == r, slab16, 0))
                    pltpu.async_copy(tab.at[sl], ring.at[slot, g * 16 + r],
                                     sems[slot])
                return carry

            lax.fori_loop(0, _CH // 16, body, 0)

        def drain(s):
            pltpu.make_async_copy(
                ut_hbm.at[pl.ds(0, _CH)], ring.at[s % 2], sems[s % 2]).wait()

        def extract(s):
            lanes = ulane_v if s < _NSTEP else ilane_v
            c = s % _NSTEP
            slot = s % 2

            def grp(g, carry):
                i16 = g * 16 + iota16
                lane16 = lanes[pl.ds(c * _CH + g * 16, 16)]
                orow16 = lax.shift_right_logical(i16, 2)
                ocol0 = (i16 & 3) * EMBED
                for j in range(EMBED):
                    w = plsc.load_gather(
                        ring.at[slot], [i16, lane16, iota16 * 0 + j])
                    plsc.store_scatter(
                        obuf.at[slot], [orow16, ocol0 + j], w)
                return carry

            lax.fori_loop(0, _CH // 16, grp, 0)

        def writeback(s):
            out = gu_out if s < _NSTEP else gi_out
            c = s % _NSTEP
            pos = pl.multiple_of((base + c * _CH) // 4, _CH // 4)
            return pltpu.async_copy(
                obuf.at[s % 2], out.at[pl.ds(pos, _CH // 4)], wsem)

        wbs = {}
        fire(0)
        for s in range(2 * _NSTEP):
            if s + 1 < 2 * _NSTEP:
                fire(s + 1)
            drain(s)
            if s >= 2:
                wbs[s - 2].wait()
            extract(s)
            wbs[s] = writeback(s)
        wbs[2 * _NSTEP - 2].wait()
        wbs[2 * _NSTEP - 1].wait()

    return k(ut3, it3, uslab, ulane, islab, ilane)


def _mlp_body(ue_ref, ie_ref, w1u_ref, w1i_ref, b1_ref, w2_ref, b2_ref,
              w3_ref, b3_ref, out_ref):
    u = ue_ref[...]
    v = ie_ref[...]
    h = (jnp.dot(u, w1u_ref[...], preferred_element_type=jnp.float32)
         + jnp.dot(v, w1i_ref[...], preferred_element_type=jnp.float32)
         + b1_ref[...])
    h = jnp.maximum(h, 0.0)
    h2 = jnp.dot(h, w2_ref[...], preferred_element_type=jnp.float32) + b2_ref[...]
    h2 = jnp.maximum(h2, 0.0)
    z = jnp.sum(h2 * w3_ref[...], axis=1) + b3_ref[0, 0]
    out_ref[...] = 1.0 / (1.0 + jnp.exp(-z))


def _tc_mlp(ue, ie, w1u, w1i, b1r, w2t, b2r, w3r, b3r):
    blk = 2048
    grid = (BATCH // blk,)
    full = lambda shape: pl.BlockSpec(shape, lambda i: (0,) * len(shape))
    return pl.pallas_call(
        _mlp_body,
        grid=grid,
        in_specs=[
            pl.BlockSpec((blk, EMBED), lambda i: (i, 0)),
            pl.BlockSpec((blk, EMBED), lambda i: (i, 0)),
            full((EMBED, 64)),
            full((EMBED, 64)),
            full((1, 64)),
            full((64, EMBED)),
            full((1, EMBED)),
            full((1, EMBED)),
            full((1, 1)),
        ],
        out_specs=pl.BlockSpec((blk,), lambda i: (i,)),
        out_shape=jax.ShapeDtypeStruct((BATCH,), jnp.float32),
    )(ue, ie, w1u, w1i, b1r, w2t, b2r, w3r, b3r)


def kernel(batch_data, user_table, item_table, W1, b1, W2, b2, W3, b3):
    uidx = batch_data[:, 0]
    iidx = batch_data[:, 1]
    ut3 = user_table.reshape(NUM_SLABS, 8, EMBED)
    it3 = item_table.reshape(NUM_SLABS, 8, EMBED)
    uslab = uidx // 8
    islab = iidx // 8
    ulane = uidx % 8
    ilane = iidx % 8
    pu, pi = _sc_gather(ut3, it3, uslab, ulane, islab, ilane)
    ue = pu.reshape(BATCH, EMBED)
    ie = pi.reshape(BATCH, EMBED)
    w1t = W1.T
    return _tc_mlp(ue, ie, w1t[:EMBED], w1t[EMBED:], b1.reshape(1, 64),
                   W2.T, b2.reshape(1, EMBED), W3, b3.reshape(1, 1))
